# baseline (speedup 1.0000x reference)
import functools

import jax
import jax.numpy as jnp
from jax import lax
from jax.experimental import pallas as pl
from jax.experimental.pallas import tpu as pltpu
from jax.experimental.pallas import tpu_sc as plsc

BATCH = 1024
INPUT_DIM = 2048
NUM_GROUPS = 8
GROUP_SIZE = 256
TB = 128
NVEC = INPUT_DIM // 16

_MSB = -2147483648


def _gumbel_noise():
    nkey = jax.random.fold_in(jax.random.key(42), 7)
    u = jax.random.uniform(nkey, (NUM_GROUPS, INPUT_DIM), dtype=jnp.float32,
                           minval=1e-7, maxval=1.0 - 1e-7)
    return -jnp.log(-jnp.log(u))


def _sc_mask(probs):
    mesh = plsc.VectorSubcoreMesh(core_axis_name="c", subcore_axis_name="s")

    @functools.partial(
        pl.kernel, mesh=mesh,
        out_type=jax.ShapeDtypeStruct((NUM_GROUPS, INPUT_DIM), jnp.float32),
        scratch_types=[
            pltpu.VMEM((INPUT_DIM,), jnp.float32),
            pltpu.VMEM((INPUT_DIM,), jnp.int32),
            pltpu.VMEM((INPUT_DIM,), jnp.float32),
            pltpu.VMEM((16,), jnp.int32),
        ],
    )
    def k(probs_hbm, mask_hbm, row_v, s_v, m_v, acc_v):
        wid = lax.axis_index("s") * 2 + lax.axis_index("c")

        @pl.when(wid < NUM_GROUPS)
        def _work():
            one = jnp.full((16,), 1, jnp.int32)
            zero = jnp.zeros((16,), jnp.int32)
            pltpu.sync_copy(probs_hbm.at[wid], row_v)

            def emb_body(i, carry):
                b = lax.bitcast_convert_type(row_v[pl.ds(i * 16, 16)],
                                             jnp.int32)
                s_v[pl.ds(i * 16, 16)] = jnp.where(
                    b >= 0, b, b ^ jnp.int32(0x7FFFFFFF))
                return carry

            lax.fori_loop(0, NVEC, emb_body, jnp.int32(0))

            def lane_total(vec):
                t = vec[0]
                for j in range(1, 16):
                    t = t + vec[j]
                return t

            def count_pred(tvec, strict):
                def body(i, acc):
                    v = s_v[pl.ds(i * 16, 16)]
                    p = jnp.where(strict, v > tvec, v >= tvec)
                    return acc + jnp.where(p, one, zero)

                return lane_total(
                    lax.fori_loop(0, NVEC, body, zero))

            def bit_body(t, tu):
                cand = tu | (jnp.int32(1) << (jnp.int32(31) - t))
                tvec = jnp.full((16,), cand ^ jnp.int32(_MSB), jnp.int32)
                cnt = count_pred(tvec, jnp.bool_(False))
                return jnp.where(cnt >= GROUP_SIZE, cand, tu)

            tu = lax.fori_loop(0, 32, bit_body, jnp.int32(0))
            t_s = tu ^ jnp.int32(_MSB)
            tvec = jnp.full((16,), t_s, jnp.int32)

            cnt_ge = count_pred(tvec, jnp.bool_(False))

            @pl.when(cnt_ge == GROUP_SIZE)
            def _no_ties():
                def body(i, carry):
                    v = s_v[pl.ds(i * 16, 16)]
                    m_v[pl.ds(i * 16, 16)] = jnp.where(
                        v >= tvec, jnp.float32(1.0), jnp.float32(0.0))
                    return carry

                lax.fori_loop(0, NVEC, body, jnp.int32(0))

            @pl.when(cnt_ge != GROUP_SIZE)
            def _ties():
                cnt_gt = count_pred(tvec, jnp.bool_(True))
                need_eq = jnp.int32(GROUP_SIZE) - cnt_gt

                def cnt_eq_le(m):
                    mvec = jnp.full((16,), m, jnp.int32)

                    def body(i, acc):
                        v = s_v[pl.ds(i * 16, 16)]
                        gidx = lax.iota(jnp.int32, 16) + jnp.full(
                            (16,), i * 16, jnp.int32)
                        p = (v == tvec) & (gidx <= mvec)
                        return acc + jnp.where(p, one, zero)

                    return lane_total(
                        lax.fori_loop(0, NVEC, body, zero))

                def search_body(t, lohi):
                    lo, hi = lohi
                    mid = (lo + hi) // 2
                    c = cnt_eq_le(mid)
                    take = c >= need_eq
                    return (jnp.where(take, lo, mid + 1),
                            jnp.where(take, mid, hi))

                lo, _ = lax.fori_loop(0, 11, search_body,
                                      (jnp.int32(0),
                                       jnp.int32(INPUT_DIM - 1)))
                mvec = jnp.full((16,), lo, jnp.int32)

                def body(i, carry):
                    v = s_v[pl.ds(i * 16, 16)]
                    gidx = lax.iota(jnp.int32, 16) + jnp.full(
                        (16,), i * 16, jnp.int32)
                    sel = (v > tvec) | ((v == tvec) & (gidx <= mvec))
                    m_v[pl.ds(i * 16, 16)] = jnp.where(
                        sel, jnp.float32(1.0), jnp.float32(0.0))
                    return carry

                lax.fori_loop(0, NVEC, body, jnp.int32(0))

            pltpu.sync_copy(m_v, mask_hbm.at[wid])

    return k(probs)


def _bcast_kernel(x_ref, mask_ref, grouped_ref):
    grouped_ref[...] = mask_ref[...][:, None, :] * x_ref[...][None, :, :]


def kernel(x, group_logits):
    probs = jax.nn.softmax((group_logits + _gumbel_noise()) / 1.0, axis=-1)
    mask = _sc_mask(probs)
    grouped = pl.pallas_call(
        _bcast_kernel,
        grid=(BATCH // TB,),
        in_specs=[
            pl.BlockSpec((TB, INPUT_DIM), lambda i: (i, 0)),
            pl.BlockSpec((NUM_GROUPS, INPUT_DIM), lambda i: (0, 0)),
        ],
        out_specs=pl.BlockSpec((NUM_GROUPS, TB, INPUT_DIM),
                               lambda i: (0, i, 0)),
        out_shape=jax.ShapeDtypeStruct((NUM_GROUPS, BATCH, INPUT_DIM),
                                       jnp.float32),
        compiler_params=pltpu.CompilerParams(
            dimension_semantics=("arbitrary",),
        ),
    )(x, mask)
    return (grouped, mask)


# speedup vs baseline: 2.0179x; 2.0179x over previous
import jax
import jax.numpy as jnp
from jax.experimental import pallas as pl
from jax.experimental.pallas import tpu as pltpu

BATCH = 1024
INPUT_DIM = 2048
NUM_GROUPS = 8
GROUP_SIZE = 256
TB = 128

_MSB = -2147483648


def _gumbel_noise():
    nkey = jax.random.fold_in(jax.random.key(42), 7)
    u = jax.random.uniform(nkey, (NUM_GROUPS, INPUT_DIM), dtype=jnp.float32,
                           minval=1e-7, maxval=1.0 - 1e-7)
    return -jnp.log(-jnp.log(u))


def _mask_kernel(probs_ref, mask_ref):
    if True:
        msb = jnp.int32(_MSB)
        z = probs_ref[...]
        b = jax.lax.bitcast_convert_type(z, jnp.int32)
        s = jnp.where(b >= 0, b, b ^ jnp.int32(0x7FFFFFFF))
        kk = jnp.int32(GROUP_SIZE)

        bps = 4
        tu = jnp.zeros((NUM_GROUPS, 1), jnp.int32)
        cnt_acc = jnp.full((NUM_GROUPS, 1), INPUT_DIM, jnp.int32)
        for low in range(32 - bps, -1, -bps):
            cands, cnts = [tu], [cnt_acc]
            for v in range(1, 1 << bps):
                shifted = v << low
                if shifted >= 2 ** 31:
                    shifted -= 2 ** 32
                cand = tu | jnp.int32(shifted)
                n = jnp.sum((s >= (cand ^ msb)).astype(jnp.int32), axis=-1,
                            keepdims=True)
                ok = n >= kk
                cands.append(jnp.where(ok, cand, tu))
                cnts.append(jnp.where(ok, n, cnt_acc))
            while len(cands) > 1:
                nc, nn = [], []
                for i in range(0, len(cands) - 1, 2):
                    take = (cands[i + 1] ^ msb) >= (cands[i] ^ msb)
                    nc.append(jnp.where(take, cands[i + 1], cands[i]))
                    nn.append(jnp.where(take, cnts[i + 1], cnts[i]))
                if len(cands) % 2:
                    nc.append(cands[-1])
                    nn.append(cnts[-1])
                cands, cnts = nc, nn
            tu, cnt_acc = cands[0], cnts[0]
        t_s = tu ^ msb

        @pl.when(jnp.all(cnt_acc == kk))
        def _no_ties():
            mask_ref[...] = (s >= t_s).astype(jnp.float32)

        @pl.when(jnp.logical_not(jnp.all(cnt_acc == kk)))
        def _break_ties():
            gt = s > t_s
            cnt_gt = jnp.sum(gt.astype(jnp.int32), axis=-1, keepdims=True)
            need_eq = GROUP_SIZE - cnt_gt
            eq = s == t_s
            idx = jax.lax.broadcasted_iota(
                jnp.int32, (NUM_GROUPS, INPUT_DIM), 1)
            lo = jnp.zeros((NUM_GROUPS, 1), jnp.int32)
            hi = jnp.full((NUM_GROUPS, 1), INPUT_DIM - 1, jnp.int32)
            for _ in range(11):
                mid = (lo + hi) // 2
                c = jnp.sum((eq & (idx <= mid)).astype(jnp.int32), axis=-1,
                            keepdims=True)
                take = c >= need_eq
                hi = jnp.where(take, mid, hi)
                lo = jnp.where(take, lo, mid + 1)
            mask_ref[...] = (gt | (eq & (idx <= lo))).astype(jnp.float32)


def _bcast_kernel(x_ref, mask_ref, grouped_ref):
    grouped_ref[...] = mask_ref[...][:, None, :] * x_ref[...][None, :, :]


def kernel(x, group_logits):
    probs = jax.nn.softmax((group_logits + _gumbel_noise()) / 1.0, axis=-1)
    mask = pl.pallas_call(
        _mask_kernel,
        out_shape=jax.ShapeDtypeStruct((NUM_GROUPS, INPUT_DIM), jnp.float32),
    )(probs)
    grouped = pl.pallas_call(
        _bcast_kernel,
        grid=(BATCH // TB,),
        in_specs=[
            pl.BlockSpec((TB, INPUT_DIM), lambda i: (i, 0)),
            pl.BlockSpec((NUM_GROUPS, INPUT_DIM), lambda i: (0, 0)),
        ],
        out_specs=pl.BlockSpec((NUM_GROUPS, TB, INPUT_DIM),
                               lambda i: (0, i, 0)),
        out_shape=jax.ShapeDtypeStruct((NUM_GROUPS, BATCH, INPUT_DIM),
                                       jnp.float32),
        compiler_params=pltpu.CompilerParams(
            dimension_semantics=("parallel",),
        ),
    )(x, mask)
    return (grouped, mask)


# speedup vs baseline: 2.1271x; 1.0542x over previous
import jax
import jax.numpy as jnp
from jax.experimental import pallas as pl
from jax.experimental.pallas import tpu as pltpu

BATCH = 1024
INPUT_DIM = 2048
NUM_GROUPS = 8
GROUP_SIZE = 256
TB = 128

_MSB = -2147483648


def _gumbel_noise():
    nkey = jax.random.fold_in(jax.random.key(42), 7)
    u = jax.random.uniform(nkey, (NUM_GROUPS, INPUT_DIM), dtype=jnp.float32,
                           minval=1e-7, maxval=1.0 - 1e-7)
    return -jnp.log(-jnp.log(u))


def _fused_kernel(x_ref, probs_ref, grouped_ref, mask_ref):
    @pl.when(pl.program_id(0) == 0)
    def _compute_mask():
        msb = jnp.int32(_MSB)
        z = probs_ref[...]
        b = jax.lax.bitcast_convert_type(z, jnp.int32)
        s = jnp.where(b >= 0, b, b ^ jnp.int32(0x7FFFFFFF))
        kk = jnp.int32(GROUP_SIZE)

        bps = 4
        tu = jnp.zeros((NUM_GROUPS, 1), jnp.int32)
        cnt_acc = jnp.full((NUM_GROUPS, 1), INPUT_DIM, jnp.int32)
        for low in range(32 - bps, -1, -bps):
            cands, cnts = [tu], [cnt_acc]
            for v in range(1, 1 << bps):
                shifted = v << low
                if shifted >= 2 ** 31:
                    shifted -= 2 ** 32
                cand = tu | jnp.int32(shifted)
                n = jnp.sum((s >= (cand ^ msb)).astype(jnp.int32), axis=-1,
                            keepdims=True)
                ok = n >= kk
                cands.append(jnp.where(ok, cand, tu))
                cnts.append(jnp.where(ok, n, cnt_acc))
            while len(cands) > 1:
                nc, nn = [], []
                for i in range(0, len(cands) - 1, 2):
                    take = (cands[i + 1] ^ msb) >= (cands[i] ^ msb)
                    nc.append(jnp.where(take, cands[i + 1], cands[i]))
                    nn.append(jnp.where(take, cnts[i + 1], cnts[i]))
                if len(cands) % 2:
                    nc.append(cands[-1])
                    nn.append(cnts[-1])
                cands, cnts = nc, nn
            tu, cnt_acc = cands[0], cnts[0]
        t_s = tu ^ msb

        @pl.when(jnp.all(cnt_acc == kk))
        def _no_ties():
            mask_ref[...] = (s >= t_s).astype(jnp.float32)

        @pl.when(jnp.logical_not(jnp.all(cnt_acc == kk)))
        def _break_ties():
            gt = s > t_s
            cnt_gt = jnp.sum(gt.astype(jnp.int32), axis=-1, keepdims=True)
            need_eq = GROUP_SIZE - cnt_gt
            eq = s == t_s
            idx = jax.lax.broadcasted_iota(
                jnp.int32, (NUM_GROUPS, INPUT_DIM), 1)
            lo = jnp.zeros((NUM_GROUPS, 1), jnp.int32)
            hi = jnp.full((NUM_GROUPS, 1), INPUT_DIM - 1, jnp.int32)
            for _ in range(11):
                mid = (lo + hi) // 2
                c = jnp.sum((eq & (idx <= mid)).astype(jnp.int32), axis=-1,
                            keepdims=True)
                take = c >= need_eq
                hi = jnp.where(take, mid, hi)
                lo = jnp.where(take, lo, mid + 1)
            mask_ref[...] = (gt | (eq & (idx <= lo))).astype(jnp.float32)

    grouped_ref[...] = mask_ref[...][:, None, :] * x_ref[...][None, :, :]


def kernel(x, group_logits):
    probs = jax.nn.softmax((group_logits + _gumbel_noise()) / 1.0, axis=-1)
    grouped, mask = pl.pallas_call(
        _fused_kernel,
        grid=(BATCH // TB,),
        in_specs=[
            pl.BlockSpec((TB, INPUT_DIM), lambda i: (i, 0)),
            pl.BlockSpec((NUM_GROUPS, INPUT_DIM), lambda i: (0, 0)),
        ],
        out_specs=[
            pl.BlockSpec((NUM_GROUPS, TB, INPUT_DIM), lambda i: (0, i, 0)),
            pl.BlockSpec((NUM_GROUPS, INPUT_DIM), lambda i: (0, 0)),
        ],
        out_shape=[
            jax.ShapeDtypeStruct((NUM_GROUPS, BATCH, INPUT_DIM), jnp.float32),
            jax.ShapeDtypeStruct((NUM_GROUPS, INPUT_DIM), jnp.float32),
        ],
        compiler_params=pltpu.CompilerParams(
            dimension_semantics=("arbitrary",),
        ),
    )(x, probs)
    return (grouped, mask)
